# baseline (speedup 1.0000x reference)
import jax
import jax.numpy as jnp
from jax.experimental import pallas as pl


def _codebook_kernel(x_ref, c_ref, out_ref):
    cb = c_ref[...]
    half_norm = 0.5 * jnp.sum(cb * cb, axis=1, keepdims=True)
    k = cb.shape[0]
    rows = []
    for b in range(x_ref.shape[0]):
        xb = x_ref[b]
        v = half_norm - jnp.dot(cb, xb,
                                preferred_element_type=jnp.float32,
                                precision=jax.lax.Precision.HIGHEST)
        best = jnp.min(v, axis=0, keepdims=True)
        iota = jax.lax.broadcasted_iota(jnp.int32, v.shape, 0)
        rows.append(jnp.min(jnp.where(v == best, iota, k), axis=0))
    out_ref[...] = jnp.stack(rows)


def kernel(x, cluster_centers):
    b, c, h, w = x.shape
    k = cluster_centers.shape[1]
    xr = x.reshape(b, c, h * w)
    cc = cluster_centers.reshape(k, c)

    idx = pl.pallas_call(
        _codebook_kernel,
        out_shape=jax.ShapeDtypeStruct((b, h * w), jnp.int32),
    )(xr, cc)
    return idx.reshape(b, 1, h, w)
